# baseline (speedup 1.0000x reference)
import jax
import jax.numpy as jnp
from jax.experimental import pallas as pl

_DIST = 100.0
_NS = 192
_NB = 16
_C = 50
_RBLK = 128


def _flipT(x, eye):
    return jax.lax.dot_general(
        x, eye, (((0,), (0,)), ((), ())),
        precision=jax.lax.Precision.HIGHEST,
        preferred_element_type=jnp.float32)


def _body(raw4_ref, z_ref, inter_ref, rd_ref, out_ref):
    raw_ref = raw4_ref.at[0]
    r_blk = _RBLK
    ii = jax.lax.broadcasted_iota(jnp.int32, (r_blk, r_blk), 0)
    jj = jax.lax.broadcasted_iota(jnp.int32, (r_blk, r_blk), 1)
    eye_r = (ii == jj).astype(jnp.float32)

    zt = _flipT(z_ref[0], eye_r)
    intert = _flipT(inter_ref[0], eye_r)
    rdt = _flipT(rd_ref[0], eye_r)

    mlt = jnp.concatenate(
        [intert[4 * b + 3:4 * b + 4, :] for b in range(_NB)], axis=0)
    mlt = jnp.where(mlt == 39.0, 41.0, mlt)
    mlt = jnp.where((mlt >= 27.0) & (mlt <= 31.0), 26.0, mlt)
    mlt = jnp.where(mlt == 9.0, 8.0, mlt)
    mlt = jnp.where(mlt == 35.0, 13.0, mlt)
    mli = mlt.astype(jnp.int32)

    inb = []
    anyin = jnp.zeros(zt.shape, dtype=jnp.bool_)
    bound_any = jnp.zeros(zt.shape, dtype=jnp.bool_)
    for b in range(_NB):
        nb = intert[4 * b:4 * b + 1, :]
        fb = intert[4 * b + 1:4 * b + 2, :]
        ib = (zt > nb) & (zt < fb)
        inb.append(ib)
        anyin = anyin | ib
        d1 = zt - fb
        d2 = nb - zt
        bound_any = bound_any | ((d1 < 0.001) & (d1 > 0.0)) | ((d2 > 0.0) & (d2 < 0.001))
    mask_bbox = (zt < _DIST) & jnp.logical_not(anyin)
    mask_bg = (zt > _DIST) & jnp.logical_not(anyin)

    densityt = raw_ref[:, :, 3].T
    s_iota = jax.lax.broadcasted_iota(jnp.int32, zt.shape, 0)
    kill = mask_bbox | bound_any | (mask_bg & (s_iota < _NS - 5))
    densityt = jnp.where(kill, 0.0, densityt)

    scale = jnp.sqrt(jnp.sum(rdt * rdt, axis=0, keepdims=True))
    zst = zt / scale
    dists = jnp.concatenate(
        [zst[1:, :] - zst[:-1, :], jnp.full((1, r_blk), 1e10, jnp.float32)],
        axis=0)
    alpha = 1.0 - jnp.exp(-jax.nn.relu(densityt) * dists)
    lt = jnp.log(1.0 - alpha + 1e-10)
    tri = (jax.lax.broadcasted_iota(jnp.int32, (_NS, _NS), 1)
           < jax.lax.broadcasted_iota(jnp.int32, (_NS, _NS), 0)).astype(jnp.float32)
    trans = jnp.exp(jax.lax.dot(tri, lt, preferred_element_type=jnp.float32))
    w = alpha * trans

    c_iota = jax.lax.broadcasted_iota(jnp.int32, (_C, r_blk), 0)
    lm = jnp.zeros((_C, r_blk), jnp.float32)
    for b in range(_NB):
        eff = inb[b]
        for bp in range(b):
            same = mli[bp:bp + 1, :] == mli[b:b + 1, :]
            eff = eff & jnp.logical_not(inb[bp] & same)
        t_b = jnp.sum(jnp.where(eff, w, 0.0), axis=0, keepdims=True)
        lm = lm + jnp.where(mli[b:b + 1, :] == c_iota, t_b, 0.0)
    t0 = jnp.sum(jnp.where(mask_bbox, w, 0.0), axis=0, keepdims=True)
    t23 = jnp.sum(jnp.where(mask_bg, w, 0.0), axis=0, keepdims=True)
    lm = lm + jnp.where(c_iota == 0, t0, 0.0)
    lm = lm + jnp.where(c_iota == 23, t23, 0.0)

    eye_c = (jax.lax.broadcasted_iota(jnp.int32, (_C, _C), 0)
             == jax.lax.broadcasted_iota(jnp.int32, (_C, _C), 1)).astype(jnp.float32)
    out_ref[:, 3 + _C:3 + 2 * _C] = _flipT(lm, eye_c)

    eye_s = (jax.lax.broadcasted_iota(jnp.int32, (_NS, _NS), 0)
             == jax.lax.broadcasted_iota(jnp.int32, (_NS, _NS), 1)).astype(jnp.float32)
    wa = _flipT(w, eye_s)
    nsc = 32
    rgb_map = jnp.zeros((r_blk, 3), jnp.float32)
    sem_map = jnp.zeros((r_blk, _C), jnp.float32)
    for s0 in range(0, _NS, nsc):
        wc = wa[:, s0:s0 + nsc, None]
        rgb_map = rgb_map + jnp.sum(
            wc * jax.nn.sigmoid(raw_ref[:, s0:s0 + nsc, 0:3]), axis=1)
        sem_map = sem_map + jnp.sum(
            wc * raw_ref[:, s0:s0 + nsc, 4:4 + _C], axis=1)
    out_ref[:, 0:3] = rgb_map
    out_ref[:, 3:3 + _C] = sem_map


def kernel(raw, z_vals, intersection, rays_d):
    b, nr, ns, ch = raw.shape
    interr = intersection.reshape(b, nr, _NB * 4)
    out = pl.pallas_call(
        _body,
        grid=(nr // _RBLK,),
        in_specs=[
            pl.BlockSpec((1, _RBLK, ns, ch), lambda i: (0, i, 0, 0)),
            pl.BlockSpec((1, _RBLK, ns), lambda i: (0, i, 0)),
            pl.BlockSpec((1, _RBLK, _NB * 4), lambda i: (0, i, 0)),
            pl.BlockSpec((1, _RBLK, 3), lambda i: (0, i, 0)),
        ],
        out_specs=pl.BlockSpec((_RBLK, 3 + 2 * _C), lambda i: (i, 0)),
        out_shape=jax.ShapeDtypeStruct((nr, 3 + 2 * _C), jnp.float32),
    )(raw, z_vals, interr, rays_d)
    return out.reshape(b, nr, 3 + 2 * _C)


# speedup vs baseline: 1.0023x; 1.0023x over previous
import jax
import jax.numpy as jnp
from jax.experimental import pallas as pl

_DIST = 100.0
_NS = 192
_NB = 16
_C = 50
_RBLK = 128


def _flipT(x, eye):
    return jax.lax.dot_general(
        x, eye, (((0,), (0,)), ((), ())),
        precision=jax.lax.Precision.HIGHEST,
        preferred_element_type=jnp.float32)


def _body(raw4_ref, z_ref, inter_ref, rd_ref, out_ref):
    raw_ref = raw4_ref.at[0]
    r_blk = _RBLK
    ii = jax.lax.broadcasted_iota(jnp.int32, (r_blk, r_blk), 0)
    jj = jax.lax.broadcasted_iota(jnp.int32, (r_blk, r_blk), 1)
    eye_r = (ii == jj).astype(jnp.float32)

    zt = _flipT(z_ref[...], eye_r)
    intert = _flipT(inter_ref[...], eye_r)
    rdt = _flipT(rd_ref[...], eye_r)

    mlt = jnp.concatenate(
        [intert[4 * b + 3:4 * b + 4, :] for b in range(_NB)], axis=0)
    mlt = jnp.where(mlt == 39.0, 41.0, mlt)
    mlt = jnp.where((mlt >= 27.0) & (mlt <= 31.0), 26.0, mlt)
    mlt = jnp.where(mlt == 9.0, 8.0, mlt)
    mlt = jnp.where(mlt == 35.0, 13.0, mlt)
    mli = mlt.astype(jnp.int32)

    inb = []
    anyin = jnp.zeros(zt.shape, dtype=jnp.bool_)
    bound_any = jnp.zeros(zt.shape, dtype=jnp.bool_)
    for b in range(_NB):
        nb = intert[4 * b:4 * b + 1, :]
        fb = intert[4 * b + 1:4 * b + 2, :]
        ib = (zt > nb) & (zt < fb)
        inb.append(ib)
        anyin = anyin | ib
        d1 = zt - fb
        d2 = nb - zt
        bound_any = bound_any | ((d1 < 0.001) & (d1 > 0.0)) | ((d2 > 0.0) & (d2 < 0.001))
    mask_bbox = (zt < _DIST) & jnp.logical_not(anyin)
    mask_bg = (zt > _DIST) & jnp.logical_not(anyin)

    densityt = raw_ref[:, :, 3].T
    s_iota = jax.lax.broadcasted_iota(jnp.int32, zt.shape, 0)
    kill = mask_bbox | bound_any | (mask_bg & (s_iota < _NS - 5))
    densityt = jnp.where(kill, 0.0, densityt)

    scale = jnp.sqrt(jnp.sum(rdt * rdt, axis=0, keepdims=True))
    zst = zt / scale
    dists = jnp.concatenate(
        [zst[1:, :] - zst[:-1, :], jnp.full((1, r_blk), 1e10, jnp.float32)],
        axis=0)
    alpha = 1.0 - jnp.exp(-jax.nn.relu(densityt) * dists)
    lt = jnp.log(1.0 - alpha + 1e-10)
    tri = (jax.lax.broadcasted_iota(jnp.int32, (_NS, _NS), 1)
           < jax.lax.broadcasted_iota(jnp.int32, (_NS, _NS), 0)).astype(jnp.float32)
    trans = jnp.exp(jax.lax.dot(tri, lt, preferred_element_type=jnp.float32))
    w = alpha * trans

    c_iota = jax.lax.broadcasted_iota(jnp.int32, (_C, r_blk), 0)
    lm = jnp.zeros((_C, r_blk), jnp.float32)
    for b in range(_NB):
        eff = inb[b]
        for bp in range(b):
            same = mli[bp:bp + 1, :] == mli[b:b + 1, :]
            eff = eff & jnp.logical_not(inb[bp] & same)
        t_b = jnp.sum(jnp.where(eff, w, 0.0), axis=0, keepdims=True)
        lm = lm + jnp.where(mli[b:b + 1, :] == c_iota, t_b, 0.0)
    t0 = jnp.sum(jnp.where(mask_bbox, w, 0.0), axis=0, keepdims=True)
    t23 = jnp.sum(jnp.where(mask_bg, w, 0.0), axis=0, keepdims=True)
    lm = lm + jnp.where(c_iota == 0, t0, 0.0)
    lm = lm + jnp.where(c_iota == 23, t23, 0.0)

    eye_c = (jax.lax.broadcasted_iota(jnp.int32, (_C, _C), 0)
             == jax.lax.broadcasted_iota(jnp.int32, (_C, _C), 1)).astype(jnp.float32)
    out_ref[:, 3 + _C:3 + 2 * _C] = _flipT(lm, eye_c)

    eye_s = (jax.lax.broadcasted_iota(jnp.int32, (_NS, _NS), 0)
             == jax.lax.broadcasted_iota(jnp.int32, (_NS, _NS), 1)).astype(jnp.float32)
    wa = _flipT(w, eye_s)
    nsc = 32
    rgb_map = jnp.zeros((r_blk, 3), jnp.float32)
    sem_map = jnp.zeros((r_blk, _C), jnp.float32)
    for s0 in range(0, _NS, nsc):
        wc = wa[:, s0:s0 + nsc, None]
        rgb_map = rgb_map + jnp.sum(
            wc * jax.nn.sigmoid(raw_ref[:, s0:s0 + nsc, 0:3]), axis=1)
        sem_map = sem_map + jnp.sum(
            wc * raw_ref[:, s0:s0 + nsc, 4:4 + _C], axis=1)
    out_ref[:, 0:3] = rgb_map
    out_ref[:, 3:3 + _C] = sem_map


def kernel(raw, z_vals, intersection, rays_d):
    b, nr, ns, ch = raw.shape
    zr = z_vals.reshape(nr, ns)
    interr = intersection.reshape(nr, _NB * 4)
    rdr = rays_d.reshape(nr, 3)
    out = pl.pallas_call(
        _body,
        grid=(nr // _RBLK,),
        in_specs=[
            pl.BlockSpec((1, _RBLK, ns, ch), lambda i: (0, i, 0, 0)),
            pl.BlockSpec((_RBLK, ns), lambda i: (i, 0)),
            pl.BlockSpec((_RBLK, _NB * 4), lambda i: (i, 0)),
            pl.BlockSpec((_RBLK, 3), lambda i: (i, 0)),
        ],
        out_specs=pl.BlockSpec((_RBLK, 3 + 2 * _C), lambda i: (i, 0)),
        out_shape=jax.ShapeDtypeStruct((nr, 3 + 2 * _C), jnp.float32),
    )(raw, zr, interr, rdr)
    return out.reshape(b, nr, 3 + 2 * _C)


# speedup vs baseline: 1.2975x; 1.2945x over previous
import jax
import jax.numpy as jnp
from jax.experimental import pallas as pl

_DIST = 100.0
_NS = 192
_NB = 16
_C = 50
_RBLK = 128


def _flipT(x, eye):
    return jax.lax.dot_general(
        x, eye, (((0,), (0,)), ((), ())),
        precision=jax.lax.Precision.HIGHEST,
        preferred_element_type=jnp.float32)


def _body(raw_ref, z_ref, inter_ref, rd_ref, out_ref):
    r_blk = _RBLK
    ii = jax.lax.broadcasted_iota(jnp.int32, (r_blk, r_blk), 0)
    jj = jax.lax.broadcasted_iota(jnp.int32, (r_blk, r_blk), 1)
    eye_r = (ii == jj).astype(jnp.float32)

    zt = _flipT(z_ref[...], eye_r)
    intert = _flipT(inter_ref[...], eye_r)
    rdt = _flipT(rd_ref[...], eye_r)

    mlt = jnp.concatenate(
        [intert[4 * b + 3:4 * b + 4, :] for b in range(_NB)], axis=0)
    mlt = jnp.where(mlt == 39.0, 41.0, mlt)
    mlt = jnp.where((mlt >= 27.0) & (mlt <= 31.0), 26.0, mlt)
    mlt = jnp.where(mlt == 9.0, 8.0, mlt)
    mlt = jnp.where(mlt == 35.0, 13.0, mlt)
    mli = mlt.astype(jnp.int32)

    inb = []
    anyin = jnp.zeros(zt.shape, dtype=jnp.bool_)
    bound_any = jnp.zeros(zt.shape, dtype=jnp.bool_)
    for b in range(_NB):
        nb = intert[4 * b:4 * b + 1, :]
        fb = intert[4 * b + 1:4 * b + 2, :]
        ib = (zt > nb) & (zt < fb)
        inb.append(ib)
        anyin = anyin | ib
        d1 = zt - fb
        d2 = nb - zt
        bound_any = bound_any | ((d1 < 0.001) & (d1 > 0.0)) | ((d2 > 0.0) & (d2 < 0.001))
    mask_bbox = (zt < _DIST) & jnp.logical_not(anyin)
    mask_bg = (zt > _DIST) & jnp.logical_not(anyin)

    densityt = raw_ref[:, :, 3].T
    s_iota = jax.lax.broadcasted_iota(jnp.int32, zt.shape, 0)
    kill = mask_bbox | bound_any | (mask_bg & (s_iota < _NS - 5))
    densityt = jnp.where(kill, 0.0, densityt)

    scale = jnp.sqrt(jnp.sum(rdt * rdt, axis=0, keepdims=True))
    zst = zt / scale
    dists = jnp.concatenate(
        [zst[1:, :] - zst[:-1, :], jnp.full((1, r_blk), 1e10, jnp.float32)],
        axis=0)
    alpha = 1.0 - jnp.exp(-jax.nn.relu(densityt) * dists)
    lt = jnp.log(1.0 - alpha + 1e-10)
    tri = (jax.lax.broadcasted_iota(jnp.int32, (_NS, _NS), 1)
           < jax.lax.broadcasted_iota(jnp.int32, (_NS, _NS), 0)).astype(jnp.float32)
    trans = jnp.exp(jax.lax.dot(tri, lt, preferred_element_type=jnp.float32))
    w = alpha * trans

    c_iota = jax.lax.broadcasted_iota(jnp.int32, (_C, r_blk), 0)
    lm = jnp.zeros((_C, r_blk), jnp.float32)
    for b in range(_NB):
        eff = inb[b]
        for bp in range(b):
            same = mli[bp:bp + 1, :] == mli[b:b + 1, :]
            eff = eff & jnp.logical_not(inb[bp] & same)
        t_b = jnp.sum(jnp.where(eff, w, 0.0), axis=0, keepdims=True)
        lm = lm + jnp.where(mli[b:b + 1, :] == c_iota, t_b, 0.0)
    t0 = jnp.sum(jnp.where(mask_bbox, w, 0.0), axis=0, keepdims=True)
    t23 = jnp.sum(jnp.where(mask_bg, w, 0.0), axis=0, keepdims=True)
    lm = lm + jnp.where(c_iota == 0, t0, 0.0)
    lm = lm + jnp.where(c_iota == 23, t23, 0.0)

    eye_c = (jax.lax.broadcasted_iota(jnp.int32, (_C, _C), 0)
             == jax.lax.broadcasted_iota(jnp.int32, (_C, _C), 1)).astype(jnp.float32)
    out_ref[:, 3 + _C:3 + 2 * _C] = _flipT(lm, eye_c)

    eye_s = (jax.lax.broadcasted_iota(jnp.int32, (_NS, _NS), 0)
             == jax.lax.broadcasted_iota(jnp.int32, (_NS, _NS), 1)).astype(jnp.float32)
    wa = _flipT(w, eye_s)
    nsc = 32
    ch_map = jnp.zeros((r_blk, 4 + _C), jnp.float32)
    for s0 in range(0, _NS, nsc):
        chunk = raw_ref[:, s0:s0 + nsc, :]
        lane3 = jax.lax.broadcasted_iota(jnp.int32, chunk.shape, 2) < 3
        val = jnp.where(lane3, jax.nn.sigmoid(chunk), chunk)
        ch_map = ch_map + jnp.sum(wa[:, s0:s0 + nsc, None] * val, axis=1)
    out_ref[:, 0:3] = ch_map[:, 0:3]
    out_ref[:, 3:3 + _C] = ch_map[:, 4:4 + _C]


def kernel(raw, z_vals, intersection, rays_d):
    b, nr, ns, ch = raw.shape
    rawr = raw.reshape(nr, ns, ch)
    zr = z_vals.reshape(nr, ns)
    interr = intersection.reshape(nr, _NB * 4)
    rdr = rays_d.reshape(nr, 3)
    out = pl.pallas_call(
        _body,
        grid=(nr // _RBLK,),
        in_specs=[
            pl.BlockSpec((_RBLK, ns, ch), lambda i: (i, 0, 0)),
            pl.BlockSpec((_RBLK, ns), lambda i: (i, 0)),
            pl.BlockSpec((_RBLK, _NB * 4), lambda i: (i, 0)),
            pl.BlockSpec((_RBLK, 3), lambda i: (i, 0)),
        ],
        out_specs=pl.BlockSpec((_RBLK, 3 + 2 * _C), lambda i: (i, 0)),
        out_shape=jax.ShapeDtypeStruct((nr, 3 + 2 * _C), jnp.float32),
    )(rawr, zr, interr, rdr)
    return out.reshape(b, nr, 3 + 2 * _C)
